# baseline (speedup 1.0000x reference)
import math

import jax
import jax.numpy as jnp
from jax.experimental import pallas as pl
from jax.experimental.pallas import tpu as pltpu

_MOMENT = 4
_BM = 384
_LANE = 128


def _cdiv(a, b):
    return -(-a // b)


def kernel(input, adj, h0, weight, w_att, lamda, alpha, l):
    n, d = input.shape
    nd = _MOMENT + 1
    alpha = jnp.asarray(alpha, jnp.float32)

    h0_2 = h0 * h0
    x = jnp.concatenate(
        [(1.0 - alpha) * input, h0, h0_2, h0_2 * h0, h0_2 * h0_2], axis=1
    ).astype(jnp.bfloat16)
    h0a = alpha * h0

    theta = math.log(1.5)
    beta = 0.9
    kw = _cdiv(n, _LANE) * _LANE
    mi = _cdiv(n, _BM)

    def body(adj_ref, x_ref, h0a_ref, w_ref, watt_ref, out_ref,
             acc0_ref, acc1_ref):
        i = pl.program_id(0)
        even = jax.lax.rem(i, 2) == 0

        def do_dot(acc_ref):
            a = adj_ref[:, 0:n].astype(jnp.bfloat16)
            acc_ref[...] = jnp.dot(a, x_ref[...],
                                   preferred_element_type=jnp.float32)

        @pl.when(jnp.logical_and(i < mi, even))
        def _dot_even():
            do_dot(acc0_ref)

        @pl.when(jnp.logical_and(i < mi, jnp.logical_not(even)))
        def _dot_odd():
            do_dot(acc1_ref)

        def epilogue(acc_ref):
            p = acc_ref[...]
            h_agg = p[:, 0:d] + h0a_ref[...]
            h_i = theta * jnp.dot(h_agg, w_ref[...],
                                  preferred_element_type=jnp.float32)
            h_i = h_i + (1.0 - theta) * h_agg

            mu = p[:, d:2 * d]
            s = p[:, 2 * d:3 * d]
            s = jnp.where(s == 0.0, 1e-16, s)
            sig = jnp.sqrt(s)
            g3 = p[:, 3 * d:4 * d]
            g3 = jnp.where(g3 == 0.0, 1e-16, g3)
            a3 = jnp.abs(g3) ** (1.0 / 3.0)
            m3 = jnp.where(g3 < 0, -a3, a3)
            g4 = p[:, 4 * d:5 * d]
            g4 = jnp.where(g4 == 0.0, 1e-16, g4)
            a4 = jnp.sqrt(jnp.sqrt(jnp.abs(g4)))
            m4 = jnp.where(g4 < 0, -a4, a4)

            wt = watt_ref[0:d, :]
            wb = watt_ref[d:2 * d, :]
            hw = jnp.dot(h_i, wb, preferred_element_type=jnp.float32)
            moms = (mu, sig, m3, m4)
            es = [
                jnp.dot(m, wt, preferred_element_type=jnp.float32) + hw
                for m in moms
            ]
            es = [jnp.where(e > 0, e, jnp.exp(e) - 1.0) for e in es]
            emax = jnp.maximum(jnp.maximum(es[0], es[1]),
                               jnp.maximum(es[2], es[3]))
            ws = [jnp.exp(e - emax) for e in es]
            denom = ws[0] + ws[1] + ws[2] + ws[3]
            h_m = (moms[0] * ws[0] + moms[1] * ws[1]
                   + moms[2] * ws[2] + moms[3] * ws[3]) / denom
            out_ref[...] = (1.0 - beta) * h_i + beta * h_m

        @pl.when(jnp.logical_and(i > 0, jnp.logical_not(even)))
        def _epi_odd():
            epilogue(acc0_ref)

        @pl.when(jnp.logical_and(i > 0, even))
        def _epi_even():
            epilogue(acc1_ref)

    grid = (mi + 1,)
    out = pl.pallas_call(
        body,
        grid=grid,
        in_specs=[
            pl.BlockSpec((_BM, kw),
                         lambda i: (jnp.minimum(i, mi - 1), 0)),
            pl.BlockSpec((n, nd * d), lambda i: (0, 0)),
            pl.BlockSpec((_BM, d),
                         lambda i: (jnp.maximum(i - 1, 0), 0)),
            pl.BlockSpec((d, d), lambda i: (0, 0)),
            pl.BlockSpec((2 * d, d), lambda i: (0, 0)),
        ],
        out_specs=pl.BlockSpec((_BM, d),
                               lambda i: (jnp.maximum(i - 1, 0), 0)),
        out_shape=jax.ShapeDtypeStruct((n, d), jnp.float32),
        scratch_shapes=[pltpu.VMEM((_BM, nd * d), jnp.float32),
                        pltpu.VMEM((_BM, nd * d), jnp.float32)],
        compiler_params=pltpu.CompilerParams(
            dimension_semantics=("arbitrary",)),
    )(adj, x, h0a, weight, w_att)
    return out


# speedup vs baseline: 1.0389x; 1.0389x over previous
import math

import jax
import jax.numpy as jnp
from jax.experimental import pallas as pl
from jax.experimental.pallas import tpu as pltpu

_MOMENT = 4
_BH = 192
_LANE = 128


def _cdiv(a, b):
    return -(-a // b)


def kernel(input, adj, h0, weight, w_att, lamda, alpha, l):
    n, d = input.shape
    nd = _MOMENT + 1
    alpha = jnp.asarray(alpha, jnp.float32)

    h0_2 = h0 * h0
    x = jnp.concatenate(
        [(1.0 - alpha) * input, h0, h0_2, h0_2 * h0, h0_2 * h0_2], axis=1
    ).astype(jnp.bfloat16)
    h0a = alpha * h0

    theta = math.log(1.5)
    beta = 0.9
    wb = w_att[d:2 * d, :]
    wc = theta * (weight @ wb) + (1.0 - theta) * wb

    kw = _cdiv(n, _LANE) * _LANE
    bm = 2 * _BH

    def epilogue(p, h0a_blk, w_ref, wt, wc_ref):
        h_agg = p[:, 0:d] + h0a_blk
        h_i = theta * jnp.dot(h_agg, w_ref[...],
                              preferred_element_type=jnp.float32)
        h_i = h_i + (1.0 - theta) * h_agg

        mu = p[:, d:2 * d]
        s = p[:, 2 * d:3 * d]
        s = jnp.where(s == 0.0, 1e-16, s)
        sig = jnp.sqrt(s)
        g3 = p[:, 3 * d:4 * d]
        g3 = jnp.where(g3 == 0.0, 1e-16, g3)
        a3 = jnp.abs(g3) ** (1.0 / 3.0)
        m3 = jnp.where(g3 < 0, -a3, a3)
        g4 = p[:, 4 * d:5 * d]
        g4 = jnp.where(g4 == 0.0, 1e-16, g4)
        a4 = jnp.sqrt(jnp.sqrt(jnp.abs(g4)))
        m4 = jnp.where(g4 < 0, -a4, a4)

        hw = jnp.dot(h_agg, wc_ref[...], preferred_element_type=jnp.float32)
        moms = (mu, sig, m3, m4)
        es = [
            jnp.dot(m, wt, preferred_element_type=jnp.float32) + hw
            for m in moms
        ]
        es = [jnp.where(e > 0, e, jnp.exp(e) - 1.0) for e in es]
        emax = jnp.maximum(jnp.maximum(es[0], es[1]),
                           jnp.maximum(es[2], es[3]))
        ws = [jnp.exp(e - emax) for e in es]
        denom = ws[0] + ws[1] + ws[2] + ws[3]
        h_m = (moms[0] * ws[0] + moms[1] * ws[1]
               + moms[2] * ws[2] + moms[3] * ws[3]) / denom
        return (1.0 - beta) * h_i + beta * h_m

    def body(adj_ref, x_ref, h0a_ref, w_ref, watt_ref, wc_ref, out_ref):
        wt = watt_ref[0:d, :]
        a_lo = adj_ref[0:_BH, 0:n].astype(jnp.bfloat16)
        a_hi = adj_ref[_BH:bm, 0:n].astype(jnp.bfloat16)
        p_lo = jnp.dot(a_lo, x_ref[...], preferred_element_type=jnp.float32)
        p_hi = jnp.dot(a_hi, x_ref[...], preferred_element_type=jnp.float32)
        out_ref[0:_BH, :] = epilogue(p_lo, h0a_ref[0:_BH, :],
                                     w_ref, wt, wc_ref)
        out_ref[_BH:bm, :] = epilogue(p_hi, h0a_ref[_BH:bm, :],
                                      w_ref, wt, wc_ref)

    grid = (_cdiv(n, bm),)
    out = pl.pallas_call(
        body,
        grid=grid,
        in_specs=[
            pl.BlockSpec((bm, kw), lambda i: (i, 0)),
            pl.BlockSpec((n, nd * d), lambda i: (0, 0)),
            pl.BlockSpec((bm, d), lambda i: (i, 0)),
            pl.BlockSpec((d, d), lambda i: (0, 0)),
            pl.BlockSpec((2 * d, d), lambda i: (0, 0)),
            pl.BlockSpec((d, d), lambda i: (0, 0)),
        ],
        out_specs=pl.BlockSpec((bm, d), lambda i: (i, 0)),
        out_shape=jax.ShapeDtypeStruct((n, d), jnp.float32),
        compiler_params=pltpu.CompilerParams(
            dimension_semantics=("parallel",)),
    )(adj, x, h0a, weight, w_att, wc)
    return out
